# Initial kernel scaffold; baseline (speedup 1.0000x reference)
import jax
import jax.numpy as jnp
from jax.experimental import pallas as pl


def kernel(vals, actions):
    raise NotImplementedError("write your pallas kernel here")


# speedup vs baseline: 1.0056x; 1.0056x over previous
import functools

import jax
import jax.numpy as jnp
from jax import lax
from jax.experimental import pallas as pl
from jax.experimental.pallas import tpu as pltpu
from jax.experimental.pallas import tpu_sc as plsc

BATCH = 16384
NCOL = 1000
NC = 2
NS = 16
L = 16
NW = NC * NS
ROWS_PER_W = BATCH // NW
CHUNK = 32
NBUF = 2
NCHUNK = ROWS_PER_W // CHUNK
CHUNK_ELEMS = CHUNK * NCOL
GROUPS = CHUNK // L

_mesh = plsc.VectorSubcoreMesh(
    core_axis_name="c", subcore_axis_name="s", num_cores=NC, num_subcores=NS
)


@functools.partial(
    pl.kernel,
    out_type=jax.ShapeDtypeStruct((BATCH * NCOL,), jnp.float32),
    mesh=_mesh,
    scratch_types=[
        pltpu.VMEM((ROWS_PER_W,), jnp.int32),
        pltpu.VMEM((ROWS_PER_W,), jnp.float32),
        *[pltpu.VMEM((CHUNK_ELEMS,), jnp.float32) for _ in range(NBUF)],
        *[pltpu.SemaphoreType.DMA for _ in range(NBUF)],
    ],
    compiler_params=pltpu.CompilerParams(needs_layout_passes=False),
)
def _onehot_sc(vals_hbm, actions_hbm, out_hbm, act_v, val_v, *buf_sem):
    bufs = buf_sem[:NBUF]
    sems = buf_sem[NBUF:]
    wid = lax.axis_index("s") * NC + lax.axis_index("c")
    row_base = wid * ROWS_PER_W
    pltpu.sync_copy(actions_hbm.at[pl.ds(row_base, ROWS_PER_W)], act_v)
    pltpu.sync_copy(vals_hbm.at[pl.ds(row_base, ROWS_PER_W)], val_v)

    zero16 = jnp.zeros((L,), jnp.float32)

    def zero_body(i, carry):
        for b in range(NBUF):
            bufs[b][pl.ds(i * L, L)] = zero16
        return carry

    lax.fori_loop(0, CHUNK_ELEMS // L, zero_body, 0)

    lane = lax.iota(jnp.int32, L)

    def scatter_chunk(buf, g, write_vals):
        for s in range(GROUPS):
            off = g * CHUNK + s * L
            a = act_v[pl.ds(off, L)]
            idx = (lane + s * L) * NCOL + a
            x = val_v[pl.ds(off, L)] if write_vals else zero16
            plsc.store_scatter(buf, [idx], x)

    out_base = wid * ROWS_PER_W * NCOL
    handles = [None] * NBUF
    for g in range(NCHUNK):
        b = g % NBUF
        if handles[b] is not None:
            handles[b].wait()
            scatter_chunk(bufs[b], g - NBUF, False)
        scatter_chunk(bufs[b], g, True)
        handles[b] = pltpu.async_copy(
            bufs[b],
            out_hbm.at[pl.ds(out_base + g * CHUNK_ELEMS, CHUNK_ELEMS)],
            sems[b],
        )
    for b in range(NBUF):
        if handles[b] is not None:
            handles[b].wait()


def kernel(vals, actions):
    return _onehot_sc(vals, actions).reshape(BATCH, NCOL)


# speedup vs baseline: 1.5418x; 1.5332x over previous
import functools

import jax
import jax.numpy as jnp
from jax import lax
from jax.experimental import pallas as pl
from jax.experimental.pallas import tpu as pltpu
from jax.experimental.pallas import tpu_sc as plsc

BATCH = 16384
NCOL = 1000
NC = 2
NS = 16
L = 16
NW = NC * NS
ROWS_PER_W = BATCH // NW
CHUNK = 32
NBUF = 2
NCHUNK = ROWS_PER_W // CHUNK
GROUPS = CHUNK // L

_mesh = plsc.VectorSubcoreMesh(
    core_axis_name="c", subcore_axis_name="s", num_cores=NC, num_subcores=NS
)


@functools.partial(
    pl.kernel,
    out_type=jax.ShapeDtypeStruct((BATCH, NCOL), jnp.float32),
    mesh=_mesh,
    scratch_types=[
        pltpu.VMEM((ROWS_PER_W,), jnp.int32),
        pltpu.VMEM((ROWS_PER_W,), jnp.float32),
        *[pltpu.VMEM((CHUNK, NCOL), jnp.float32) for _ in range(NBUF)],
        *[pltpu.SemaphoreType.DMA for _ in range(NBUF)],
    ],
    compiler_params=pltpu.CompilerParams(
        needs_layout_passes=False, use_tc_tiling_on_sc=True
    ),
)
def _onehot_sc(vals_hbm, actions_hbm, out_hbm, act_v, val_v, *buf_sem):
    bufs = buf_sem[:NBUF]
    sems = buf_sem[NBUF:]
    wid = lax.axis_index("s") * NC + lax.axis_index("c")
    row_base = wid * ROWS_PER_W
    pltpu.sync_copy(actions_hbm.at[pl.ds(row_base, ROWS_PER_W)], act_v)
    pltpu.sync_copy(vals_hbm.at[pl.ds(row_base, ROWS_PER_W)], val_v)

    zero16 = jnp.zeros((L,), jnp.float32)

    stores_per_row = (NCOL + L - 1) // L

    def zero_body(i, carry):
        r = i // stores_per_row
        c = jnp.minimum((i % stores_per_row) * L, NCOL - L)
        for b in range(NBUF):
            bufs[b][r, pl.ds(c, L)] = zero16
        return carry

    lax.fori_loop(0, CHUNK * stores_per_row, zero_body, 0)

    lane = lax.iota(jnp.int32, L)

    def scatter_chunk(buf, g, write_vals):
        for s in range(GROUPS):
            off = g * CHUNK + s * L
            a = act_v[pl.ds(off, L)]
            r = lane + s * L
            x = val_v[pl.ds(off, L)] if write_vals else zero16
            plsc.store_scatter(buf, [r, a], x)

    handles = [None] * NBUF
    for g in range(NCHUNK):
        b = g % NBUF
        if handles[b] is not None:
            handles[b].wait()
            scatter_chunk(bufs[b], g - NBUF, False)
        scatter_chunk(bufs[b], g, True)
        handles[b] = pltpu.async_copy(
            bufs[b],
            out_hbm.at[pl.ds(row_base + g * CHUNK, CHUNK)],
            sems[b],
        )
    for b in range(NBUF):
        if handles[b] is not None:
            handles[b].wait()


def kernel(vals, actions):
    return _onehot_sc(vals, actions)


# speedup vs baseline: 2.3552x; 1.5275x over previous
import functools

import jax
import jax.numpy as jnp
from jax import lax
from jax.experimental import pallas as pl
from jax.experimental.pallas import tpu as pltpu
from jax.experimental.pallas import tpu_sc as plsc

BATCH = 16384
NCOL = 1000
NC = 2
NS = 16
L = 16
NW = NC * NS
COLS_PER_W = BATCH // NW
CW = 128
NBUF = 1
NCHUNK = COLS_PER_W // CW
GROUPS = CW // L

_mesh = plsc.VectorSubcoreMesh(
    core_axis_name="c", subcore_axis_name="s", num_cores=NC, num_subcores=NS
)


@functools.partial(
    pl.kernel,
    out_type=jax.ShapeDtypeStruct((NCOL, BATCH), jnp.float32),
    mesh=_mesh,
    scratch_types=[
        pltpu.VMEM((COLS_PER_W,), jnp.int32),
        pltpu.VMEM((COLS_PER_W,), jnp.float32),
        *[pltpu.VMEM((NCOL, CW), jnp.float32) for _ in range(NBUF)],
        *[pltpu.SemaphoreType.DMA for _ in range(NBUF)],
    ],
    compiler_params=pltpu.CompilerParams(
        needs_layout_passes=False, use_tc_tiling_on_sc=True
    ),
)
def _onehot_sc(vals_hbm, actions_hbm, out_hbm, act_v, val_v, *buf_sem):
    bufs = buf_sem[:NBUF]
    sems = buf_sem[NBUF:]
    wid = lax.axis_index("s") * NC + lax.axis_index("c")
    col_base = wid * COLS_PER_W
    pltpu.sync_copy(actions_hbm.at[pl.ds(col_base, COLS_PER_W)], act_v)
    pltpu.sync_copy(vals_hbm.at[pl.ds(col_base, COLS_PER_W)], val_v)

    zero16 = jnp.zeros((L,), jnp.float32)

    def zero_body(i, carry):
        r = i // GROUPS
        c = (i % GROUPS) * L
        for b in range(NBUF):
            bufs[b][r, pl.ds(c, L)] = zero16
        return carry

    lax.fori_loop(0, NCOL * GROUPS, zero_body, 0)

    lane = lax.iota(jnp.int32, L)

    def scatter_chunk(buf, g, write_vals):
        for s in range(GROUPS):
            off = g * CW + s * L
            a = act_v[pl.ds(off, L)]
            c = lane + s * L
            x = val_v[pl.ds(off, L)] if write_vals else zero16
            plsc.store_scatter(buf, [a, c], x)

    handles = [None] * NBUF
    for g in range(NCHUNK):
        b = g % NBUF
        if handles[b] is not None:
            handles[b].wait()
            scatter_chunk(bufs[b], g - NBUF, False)
        scatter_chunk(bufs[b], g, True)
        handles[b] = pltpu.async_copy(
            bufs[b],
            out_hbm.at[:, pl.ds(col_base + g * CW, CW)],
            sems[b],
        )
    for b in range(NBUF):
        if handles[b] is not None:
            handles[b].wait()


def kernel(vals, actions):
    return _onehot_sc(vals, actions).T


# speedup vs baseline: 3.3333x; 1.4153x over previous
import functools

import jax
import jax.numpy as jnp
from jax import lax
from jax.experimental import pallas as pl
from jax.experimental.pallas import tpu as pltpu
from jax.experimental.pallas import tpu_sc as plsc

BATCH = 16384
NCOL = 1000
NC = 2
NS = 16
L = 16
NW = NC * NS
COLS_PER_W = BATCH // NW
CW = 128
RB = 200
NBANDS = NCOL // RB
NBUF = 2
NCHUNK = COLS_PER_W // CW
PIECES = NCHUNK * NBANDS
GROUPS = CW // L

_mesh = plsc.VectorSubcoreMesh(
    core_axis_name="c", subcore_axis_name="s", num_cores=NC, num_subcores=NS
)


@functools.partial(
    pl.kernel,
    out_type=jax.ShapeDtypeStruct((NCOL, BATCH), jnp.float32),
    mesh=_mesh,
    scratch_types=[
        pltpu.VMEM((COLS_PER_W,), jnp.int32),
        pltpu.VMEM((COLS_PER_W,), jnp.float32),
        *[pltpu.VMEM((RB, CW), jnp.float32) for _ in range(NBUF)],
        *[pltpu.SemaphoreType.DMA for _ in range(NBUF)],
    ],
    compiler_params=pltpu.CompilerParams(
        needs_layout_passes=False, use_tc_tiling_on_sc=True
    ),
)
def _onehot_sc(vals_hbm, actions_hbm, out_hbm, act_v, val_v, *buf_sem):
    bufs = buf_sem[:NBUF]
    sems = buf_sem[NBUF:]
    wid = lax.axis_index("s") * NC + lax.axis_index("c")
    col_base = wid * COLS_PER_W
    pltpu.sync_copy(actions_hbm.at[pl.ds(col_base, COLS_PER_W)], act_v)
    pltpu.sync_copy(vals_hbm.at[pl.ds(col_base, COLS_PER_W)], val_v)

    zero16 = jnp.zeros((L,), jnp.float32)

    def zero_body(i, carry):
        r = i // GROUPS
        c = (i % GROUPS) * L
        for b in range(NBUF):
            bufs[b][r, pl.ds(c, L)] = zero16
        return carry

    lax.fori_loop(0, RB * GROUPS, zero_body, 0)

    lane = lax.iota(jnp.int32, L)

    def scatter_piece(buf, p, write_vals):
        g, band = divmod(p, NBANDS)
        lo = band * RB
        for s in range(GROUPS):
            off = g * CW + s * L
            a = act_v[pl.ds(off, L)]
            m = (a >= lo) & (a < lo + RB)
            c = lane + s * L
            x = val_v[pl.ds(off, L)] if write_vals else zero16
            plsc.store_scatter(buf, [a - lo, c], x, mask=m)

    handles = [None] * NBUF
    for p in range(PIECES):
        b = p % NBUF
        if handles[b] is not None:
            handles[b].wait()
            scatter_piece(bufs[b], p - NBUF, False)
        scatter_piece(bufs[b], p, True)
        g, band = divmod(p, NBANDS)
        handles[b] = pltpu.async_copy(
            bufs[b],
            out_hbm.at[pl.ds(band * RB, RB), pl.ds(col_base + g * CW, CW)],
            sems[b],
        )
    for b in range(NBUF):
        if handles[b] is not None:
            handles[b].wait()


def kernel(vals, actions):
    return _onehot_sc(vals, actions).T
